# baseline (speedup 1.0000x reference)
import functools

import jax
import jax.numpy as jnp
from jax import lax
from jax.experimental import pallas as pl
from jax.experimental.pallas import tpu as pltpu
from jax.experimental.pallas import tpu_sc as plsc

NC = 2
NS = 16
NW = NC * NS
CHUNK = 128


@functools.lru_cache(maxsize=None)
def _build(batch, widths):
    bpw = batch // NW
    nch = bpw // CHUNK
    nbuf = len(widths)
    mesh = plsc.VectorSubcoreMesh(
        core_axis_name="c", subcore_axis_name="s",
        num_cores=NC, num_subcores=NS)

    def oshape(w):
        return (batch,) if w == 1 else (batch, w)

    def sshape(w):
        return (bpw,) if w == 1 else (bpw, w)

    out_type = tuple(
        jax.ShapeDtypeStruct(oshape(w), jnp.float32) for w in widths)
    scratch = (
        [pltpu.VMEM((nch, CHUNK), jnp.int32)]
        + [pltpu.VMEM(sshape(w), jnp.float32) for w in widths]
        + [pltpu.SemaphoreType.DMA]
    )

    @functools.partial(
        pl.kernel, out_type=out_type, scratch_types=scratch, mesh=mesh,
        compiler_params=pltpu.CompilerParams(use_tc_tiling_on_sc=False))
    def k(idx_hbm, *refs):
        tabs = refs[:nbuf]
        outs = refs[nbuf:2 * nbuf]
        idx_v = refs[2 * nbuf]
        rows = refs[2 * nbuf + 1:2 * nbuf + 1 + nbuf]
        sem = refs[-1]
        wid = lax.axis_index("s") * NC + lax.axis_index("c")
        pltpu.sync_copy(idx_hbm.at[pl.ds(wid * nch, nch)], idx_v)
        for j in range(nch):
            cps = [
                pltpu.async_copy(
                    tabs[b].at[idx_v.at[j]],
                    rows[b].at[pl.ds(j * CHUNK, CHUNK)],
                    sem)
                for b in range(nbuf)
            ]
            for c in cps:
                c.wait()
        for b in range(nbuf):
            pltpu.sync_copy(rows[b], outs[b].at[pl.ds(wid * bpw, bpw)])

    return k


def kernel(indices, obs_buffer, next_obs_buffer, acts_buffer, rewards_buffer,
           terminals_buffer, rew_vects_buffer, term_vects_buffer):
    tabs = (obs_buffer, acts_buffer, rewards_buffer, terminals_buffer,
            next_obs_buffer, rew_vects_buffer, term_vects_buffer)
    batch = indices.shape[0]
    widths = tuple(t.shape[1] for t in tabs)
    flat_tabs = tuple(t.reshape(t.shape[0]) if t.shape[1] == 1 else t
                      for t in tabs)
    idx2d = indices.reshape(batch // CHUNK, CHUNK)
    k = _build(batch, widths)
    outs = k(idx2d, *flat_tabs)
    return tuple(o.reshape(batch, w) if w == 1 else o
                 for o, w in zip(outs, widths))
